# baseline (speedup 1.0000x reference)
import functools

import jax
import jax.numpy as jnp
from jax import lax
from jax.experimental import pallas as pl
from jax.experimental.pallas import tpu as pltpu
from jax.experimental.pallas import tpu_sc as plsc

B = 200000
D_F = 128
D_E = 64
D_OUT = D_F + D_E

NC = 2
NS = 16
NW = NC * NS

BLK = 128
N_FULL = B // BLK
TAIL = B - N_FULL * BLK
BPW = (N_FULL + NW - 1) // NW

FEAT_PER_W = 6256
FEAT_LAST = B - (NW - 1) * FEAT_PER_W


def _node_prep_kernel(ids_hbm, feats_hbm, table_hbm, out_hbm,
                      idx_v, rows_v, idx_t, rows_t, sem):
    wid = lax.axis_index("s") * NC + lax.axis_index("c")

    fbase = wid * FEAT_PER_W

    @pl.when(wid < NW - 1)
    def _():
        pltpu.sync_copy(feats_hbm.at[pl.ds(fbase, FEAT_PER_W), :],
                        out_hbm.at[pl.ds(fbase, FEAT_PER_W), pl.ds(0, D_F)])

    @pl.when(wid == NW - 1)
    def _():
        lbase = (NW - 1) * FEAT_PER_W
        pltpu.sync_copy(feats_hbm.at[pl.ds(lbase, FEAT_LAST), :],
                        out_hbm.at[pl.ds(lbase, FEAT_LAST), pl.ds(0, D_F)])

    def body(i, carry):
        blk = wid * BPW + i

        @pl.when(blk < N_FULL)
        def _():
            base = blk * BLK
            pltpu.sync_copy(ids_hbm.at[pl.ds(base, BLK)], idx_v)
            pltpu.async_copy(table_hbm.at[idx_v], rows_v, sem).wait()
            pltpu.sync_copy(rows_v,
                            out_hbm.at[pl.ds(base, BLK), pl.ds(D_F, D_E)])
        return carry

    lax.fori_loop(0, BPW, body, 0)

    @pl.when(wid == NW - 1)
    def _():
        base = N_FULL * BLK
        pltpu.sync_copy(ids_hbm.at[pl.ds(base, TAIL)], idx_t)
        pltpu.async_copy(table_hbm.at[idx_t], rows_t, sem).wait()
        pltpu.sync_copy(rows_t,
                        out_hbm.at[pl.ds(base, TAIL), pl.ds(D_F, D_E)])


@jax.jit
def _node_prep(ids, feats, emb_W):
    mesh = plsc.VectorSubcoreMesh(core_axis_name="c", subcore_axis_name="s")
    run = pl.kernel(
        _node_prep_kernel,
        out_type=jax.ShapeDtypeStruct((B, D_OUT), jnp.float32),
        mesh=mesh,
        scratch_types=[
            pltpu.VMEM((BLK,), jnp.int32),
            pltpu.VMEM((BLK, D_E), jnp.float32),
            pltpu.VMEM((TAIL,), jnp.int32),
            pltpu.VMEM((TAIL, D_E), jnp.float32),
            pltpu.SemaphoreType.DMA,
        ],
        compiler_params=pltpu.CompilerParams(use_tc_tiling_on_sc=False),
    )
    return run(ids, feats, emb_W)


def kernel(ids, feats, hop_idx, emb_W):
    n_nodes = emb_W.shape[0] - 1
    ids = ids.astype(jnp.int32)
    gather_ids = jnp.where(hop_idx > 0, ids, jnp.full_like(ids, n_nodes))
    return _node_prep(gather_ids, feats, emb_W)


# speedup vs baseline: 6.9634x; 6.9634x over previous
import jax
import jax.numpy as jnp
from jax import lax
from jax.experimental import pallas as pl
from jax.experimental.pallas import tpu as pltpu
from jax.experimental.pallas import tpu_sc as plsc

B = 200000
D_F = 128
D_E = 64
D_OUT = D_F + D_E
D_P = 2 * D_E

NC = 2
NS = 16
NW = NC * NS

BLK = 128
N_FULL = B // BLK
TAIL = B - N_FULL * BLK
BPW = (N_FULL + NW - 1) // NW

ROWS_TC = 2000
N_TC = B // ROWS_TC


def _gather_kernel(ids_hbm, pairs_hbm, stage_hbm,
                   idx_v, rows_v, idx_t, rows_t, sem):
    wid = lax.axis_index("s") * NC + lax.axis_index("c")

    def body(i, carry):
        blk = wid * BPW + i

        @pl.when(blk < N_FULL)
        def _():
            base = blk * BLK
            pltpu.sync_copy(ids_hbm.at[pl.ds(base, BLK)], idx_v)
            pltpu.async_copy(pairs_hbm.at[idx_v], rows_v, sem).wait()
            pltpu.sync_copy(rows_v, stage_hbm.at[pl.ds(base, BLK)])
        return carry

    lax.fori_loop(0, BPW, body, 0)

    @pl.when(wid == NW - 1)
    def _():
        base = N_FULL * BLK
        pltpu.sync_copy(ids_hbm.at[pl.ds(base, TAIL)], idx_t)
        pltpu.async_copy(pairs_hbm.at[idx_t], rows_t, sem).wait()
        pltpu.sync_copy(rows_t, stage_hbm.at[pl.ds(base, TAIL)])


def _assemble_kernel(feats_ref, stage_ref, ids_ref, out_ref):
    out_ref[:, 0:D_F] = feats_ref[...]
    par = (ids_ref[...] & 1) == 1
    out_ref[:, D_F:D_OUT] = jnp.where(
        par, stage_ref[:, D_E:D_P], stage_ref[:, 0:D_E])


@jax.jit
def _node_prep(gather_ids, feats, emb_W):
    pairs = jnp.pad(emb_W, ((0, 1), (0, 0))).reshape(-1, D_P)
    pair_ids = lax.shift_right_logical(gather_ids, 1)

    mesh = plsc.VectorSubcoreMesh(core_axis_name="c", subcore_axis_name="s")
    gather = pl.kernel(
        _gather_kernel,
        out_type=jax.ShapeDtypeStruct((B, D_P), jnp.float32),
        mesh=mesh,
        scratch_types=[
            pltpu.VMEM((BLK,), jnp.int32),
            pltpu.VMEM((BLK, D_P), jnp.float32),
            pltpu.VMEM((TAIL,), jnp.int32),
            pltpu.VMEM((TAIL, D_P), jnp.float32),
            pltpu.SemaphoreType.DMA,
        ],
    )
    stage = gather(pair_ids, pairs)

    out = pl.pallas_call(
        _assemble_kernel,
        grid=(N_TC,),
        in_specs=[
            pl.BlockSpec((ROWS_TC, D_F), lambda i: (i, 0)),
            pl.BlockSpec((ROWS_TC, D_P), lambda i: (i, 0)),
            pl.BlockSpec((ROWS_TC, 1), lambda i: (i, 0)),
        ],
        out_specs=pl.BlockSpec((ROWS_TC, D_OUT), lambda i: (i, 0)),
        out_shape=jax.ShapeDtypeStruct((B, D_OUT), jnp.float32),
    )(feats, stage, gather_ids.reshape(B, 1))
    return out


def kernel(ids, feats, hop_idx, emb_W):
    n_nodes = emb_W.shape[0] - 1
    ids = ids.astype(jnp.int32)
    gather_ids = jnp.where(hop_idx > 0, ids, jnp.full_like(ids, n_nodes))
    return _node_prep(gather_ids, feats, emb_W)


# speedup vs baseline: 8.4763x; 1.2173x over previous
import jax
import jax.numpy as jnp
from jax import lax
from jax.experimental import pallas as pl
from jax.experimental.pallas import tpu as pltpu
from jax.experimental.pallas import tpu_sc as plsc

B = 200000
D_F = 128
D_E = 64
D_OUT = D_F + D_E
D_P = 128

NC = 2
NS = 16
NW = NC * NS

BLK = 128
N_FULL = B // BLK
TAIL = B - N_FULL * BLK
BPW = (N_FULL + NW - 1) // NW

ROWS_TC = 2000
N_TC = B // ROWS_TC


def _gather_kernel(ids_hbm, table_hbm, stage_hbm,
                   idx_v, rows_v, idx_t, rows_t, sem):
    wid = lax.axis_index("s") * NC + lax.axis_index("c")

    def body(i, carry):
        blk = wid * BPW + i

        @pl.when(blk < N_FULL)
        def _():
            base = blk * BLK
            pltpu.sync_copy(ids_hbm.at[pl.ds(base, BLK)], idx_v)
            pltpu.async_copy(table_hbm.at[idx_v], rows_v, sem).wait()
            pltpu.sync_copy(rows_v, stage_hbm.at[pl.ds(base, BLK)])
        return carry

    lax.fori_loop(0, BPW, body, 0)

    @pl.when(wid == NW - 1)
    def _():
        base = N_FULL * BLK
        pltpu.sync_copy(ids_hbm.at[pl.ds(base, TAIL)], idx_t)
        pltpu.async_copy(table_hbm.at[idx_t], rows_t, sem).wait()
        pltpu.sync_copy(rows_t, stage_hbm.at[pl.ds(base, TAIL)])


def _feats_kernel(feats_ref, out_ref):
    out_ref[...] = feats_ref[...]


def _emb_kernel(stage_ref, base_ref, out_ref):
    del base_ref
    out_ref[:, 0:D_E] = stage_ref[:, 0:D_E]


@jax.jit
def _node_prep(gather_ids, feats, emb_W):
    table = jnp.pad(emb_W, ((0, 0), (0, D_P - D_E)))

    mesh = plsc.VectorSubcoreMesh(core_axis_name="c", subcore_axis_name="s")
    gather = pl.kernel(
        _gather_kernel,
        out_type=jax.ShapeDtypeStruct((B, D_P), jnp.float32),
        mesh=mesh,
        scratch_types=[
            pltpu.VMEM((BLK,), jnp.int32),
            pltpu.VMEM((BLK, D_P), jnp.float32),
            pltpu.VMEM((TAIL,), jnp.int32),
            pltpu.VMEM((TAIL, D_P), jnp.float32),
            pltpu.SemaphoreType.DMA,
        ],
    )
    stage = gather(gather_ids, table)

    base = pl.pallas_call(
        _feats_kernel,
        grid=(N_TC,),
        in_specs=[pl.BlockSpec((ROWS_TC, D_F), lambda i: (i, 0))],
        out_specs=pl.BlockSpec((ROWS_TC, D_F), lambda i: (i, 0)),
        out_shape=jax.ShapeDtypeStruct((B, D_OUT), jnp.float32),
    )(feats)

    out = pl.pallas_call(
        _emb_kernel,
        grid=(N_TC,),
        in_specs=[
            pl.BlockSpec((ROWS_TC, D_P), lambda i: (i, 0)),
            pl.BlockSpec(memory_space=pl.ANY),
        ],
        out_specs=pl.BlockSpec((ROWS_TC, D_F), lambda i: (i, 1)),
        out_shape=jax.ShapeDtypeStruct((B, D_OUT), jnp.float32),
        input_output_aliases={1: 0},
    )(stage, base)
    return out


def kernel(ids, feats, hop_idx, emb_W):
    n_nodes = emb_W.shape[0] - 1
    ids = ids.astype(jnp.int32)
    gather_ids = jnp.where(hop_idx > 0, ids, jnp.full_like(ids, n_nodes))
    return _node_prep(gather_ids, feats, emb_W)
